# baseline (speedup 1.0000x reference)
import functools

import jax
import jax.numpy as jnp
from jax import lax
from jax.experimental import pallas as pl
from jax.experimental.pallas import tpu as pltpu
from jax.experimental.pallas import tpu_sc as plsc

D = 64
NC, NS = 2, 16
NW = NC * NS
CHUNK = 128
NBUF = 4
SHIFT = 14

PB = 16384
CB = 2 * PB
UPR = 1024


def _repack_body(t_ref, o_ref):
    blk = t_ref[...]
    a = blk[:, :PB].T
    b = blk[:, PB:].T
    o_ref[...] = jnp.concatenate([a, b], axis=1)


def _repack(table_t, vocab):
    nblk = pl.cdiv(vocab, CB)
    out2 = pl.pallas_call(
        _repack_body,
        grid=(nblk,),
        in_specs=[pl.BlockSpec((D, CB), lambda i: (0, i))],
        out_specs=pl.BlockSpec((PB, 128), lambda i: (i, 0)),
        out_shape=jax.ShapeDtypeStruct((nblk * PB, 128), jnp.float32),
    )(table_t)
    return out2.reshape(nblk * PB * 2, D)


@functools.lru_cache(maxsize=None)
def _make_gather(B, vpad):
    assert B % (NW * CHUNK * NBUF) == 0
    b_per_w = B // NW
    n_chunks = b_per_w // CHUNK
    mesh = plsc.VectorSubcoreMesh(core_axis_name="c", subcore_axis_name="s")

    @functools.partial(
        pl.kernel,
        mesh=mesh,
        out_type=jax.ShapeDtypeStruct((B, D), jnp.float32),
        compiler_params=pltpu.CompilerParams(use_tc_tiling_on_sc=False),
        scratch_types=[
            pltpu.VMEM((b_per_w,), jnp.int32),
            pltpu.VMEM((NBUF, CHUNK, D), jnp.float32),
            pltpu.SemaphoreType.DMA((NBUF,)),
        ],
    )
    def k(idx_hbm, table_hbm, out_hbm, idx_v, bufs, sems):
        wid = lax.axis_index("s") * NC + lax.axis_index("c")
        base = wid * b_per_w
        pltpu.sync_copy(idx_hbm.at[pl.ds(base, b_per_w)], idx_v)

        def remap(j, carry):
            v = idx_v[pl.ds(j * 16, 16)]
            lin = (v & ~(2 * PB - 1)) | ((v & (PB - 1)) << 1) | ((v >> SHIFT) & 1)
            idx_v[pl.ds(j * 16, 16)] = lin
            return carry

        lax.fori_loop(0, b_per_w // 16, remap, 0)

        def gather(i, b):
            pltpu.make_async_copy(
                table_hbm.at[idx_v.at[pl.ds(i * CHUNK, CHUNK)]],
                bufs.at[b],
                sems.at[b],
            ).start()

        for b in range(NBUF):
            gather(b, b)

        def body(g, carry):
            c = g * NBUF
            for b in range(NBUF):
                i = c + b
                pltpu.make_async_copy(
                    table_hbm.at[idx_v.at[pl.ds(0, CHUNK)]],
                    bufs.at[b],
                    sems.at[b],
                ).wait()
                pltpu.sync_copy(
                    bufs.at[b], out_hbm.at[pl.ds(base + i * CHUNK, CHUNK)]
                )
                nxt = i + NBUF

                @pl.when(nxt < n_chunks)
                def _():
                    gather(nxt, b)

            return carry

        lax.fori_loop(0, n_chunks // NBUF, body, 0)

    return k


def _unpack_body(z_ref, p_ref):
    blk = z_ref[0]
    for s in range(UPR // 64):
        rows = blk[64 * s : 64 * (s + 1), :]
        a = rows[:, :D].T
        b = rows[:, D:].T
        c = jnp.stack([a, b], axis=-1).reshape(D, 128)
        p_ref[0, :, 128 * s : 128 * (s + 1)] = c


def _unpack(z3, rows, cols):
    nw = cols // (2 * UPR)
    return pl.pallas_call(
        _unpack_body,
        grid=(rows, nw),
        in_specs=[pl.BlockSpec((1, UPR, 128), lambda j, w: (j, w, 0))],
        out_specs=pl.BlockSpec((1, D, 2 * UPR), lambda j, w: (j, 0, w)),
        out_shape=jax.ShapeDtypeStruct((rows, D, cols), jnp.float32),
    )(z3)


@jax.jit
def kernel(x, table):
    r, c = x.shape
    B = r * c
    vocab = table.shape[0]
    x_flat = x.T.reshape(B).astype(jnp.int32)
    table_lin = _repack(table.T, vocab)
    out = _make_gather(B, table_lin.shape[0])(x_flat, table_lin)
    z3 = out.reshape(c, r // 2, 2 * D)
    p = _unpack(z3, c, r)
    return p.transpose(2, 0, 1)


# speedup vs baseline: 11.5253x; 11.5253x over previous
import functools

import jax
import jax.numpy as jnp
from jax import lax
from jax.experimental import pallas as pl
from jax.experimental.pallas import tpu as pltpu
from jax.experimental.pallas import tpu_sc as plsc

D = 64
NC, NS = 2, 16
NW = NC * NS
CHUNK = 128
NBUF = 4
SHIFT = 14

PB = 16384
CB = 2 * PB
UPR = 1024


def _repack_body(t_ref, o_ref):
    blk = t_ref[...]
    a = blk[:, :PB].T
    b = blk[:, PB:].T
    o_ref[...] = jnp.concatenate([a, b], axis=1)


def _repack(table_t, vocab):
    nblk = pl.cdiv(vocab, CB)
    out2 = pl.pallas_call(
        _repack_body,
        grid=(nblk,),
        in_specs=[pl.BlockSpec((D, CB), lambda i: (0, i))],
        out_specs=pl.BlockSpec((PB, 128), lambda i: (i, 0)),
        out_shape=jax.ShapeDtypeStruct((nblk * PB, 128), jnp.float32),
    )(table_t)
    return out2.reshape(nblk * PB * 2, D)


def _unpack_body(z_ref, p_ref):
    blk = z_ref[0]
    for s in range(UPR // 128):
        c = blk[128 * s : 128 * (s + 1), :].T
        p_ref[:, 0, :, 128 * s : 128 * (s + 1)] = c.reshape(2, D, 128)


def _unpack(z3, half, rows):
    nw = rows // UPR
    return pl.pallas_call(
        _unpack_body,
        grid=(half, nw),
        in_specs=[pl.BlockSpec((1, UPR, 128), lambda j, w: (j, w, 0))],
        out_specs=pl.BlockSpec((2, 1, D, UPR), lambda j, w: (0, j, 0, w)),
        out_shape=jax.ShapeDtypeStruct((2, half, D, rows), jnp.float32),
    )(z3)


@functools.lru_cache(maxsize=None)
def _make_gather(B, vpad):
    assert B % (NW * CHUNK * NBUF) == 0
    b_per_w = B // NW
    n_chunks = b_per_w // CHUNK
    mesh = plsc.VectorSubcoreMesh(core_axis_name="c", subcore_axis_name="s")

    @functools.partial(
        pl.kernel,
        mesh=mesh,
        out_type=jax.ShapeDtypeStruct((B, D), jnp.float32),
        compiler_params=pltpu.CompilerParams(use_tc_tiling_on_sc=False),
        scratch_types=[
            pltpu.VMEM((b_per_w,), jnp.int32),
            pltpu.VMEM((n_chunks, CHUNK), jnp.int32),
            pltpu.VMEM((NBUF, CHUNK, D), jnp.float32),
            pltpu.SemaphoreType.DMA((NBUF,)),
            pltpu.SemaphoreType.DMA,
        ],
    )
    def k(idx_hbm, table_hbm, out_hbm, idx_v, pos_v, bufs, sems, sem_s):
        wid = lax.axis_index("s") * NC + lax.axis_index("c")
        base = wid * b_per_w
        pltpu.sync_copy(idx_hbm.at[pl.ds(base, b_per_w)], idx_v)

        lane = lax.iota(jnp.int32, 16)

        def remap(g, carry):
            v = idx_v[pl.ds(g * 16, 16)]
            lin = (v & ~(2 * PB - 1)) | ((v & (PB - 1)) << 1) | ((v >> SHIFT) & 1)
            idx_v[pl.ds(g * 16, 16)] = lin
            b = base + g * 16 + lane
            j = b >> 14
            i = b & 16383
            h = jnp.where(j >= 10, 1, 0)
            pos = (j - 10 * h) * 32768 + 2 * i + h
            pos_v[g // 8, pl.ds(16 * (g % 8), 16)] = pos
            return carry

        lax.fori_loop(0, b_per_w // 16, remap, 0)

        def gather(i, b):
            pltpu.make_async_copy(
                table_hbm.at[idx_v.at[pl.ds(i * CHUNK, CHUNK)]],
                bufs.at[b],
                sems.at[b],
            ).start()

        for b in range(NBUF):
            gather(b, b)

        def body(g, carry):
            c = g * NBUF
            for b in range(NBUF):
                i = c + b
                pltpu.make_async_copy(
                    table_hbm.at[idx_v.at[pl.ds(0, CHUNK)]],
                    bufs.at[b],
                    sems.at[b],
                ).wait()
                pltpu.async_copy(
                    bufs.at[b], out_hbm.at[pos_v.at[i]], sem_s
                ).wait()
                nxt = i + NBUF

                @pl.when(nxt < n_chunks)
                def _():
                    gather(nxt, b)

            return carry

        lax.fori_loop(0, n_chunks // NBUF, body, 0)

    return k


@jax.jit
def kernel(x, table):
    r, c = x.shape
    B = r * c
    vocab = table.shape[0]
    x_flat = x.T.reshape(B).astype(jnp.int32)
    table_lin = _repack(table.T, vocab)
    out = _make_gather(B, table_lin.shape[0])(x_flat, table_lin)
    z3 = out.reshape(c // 2, r, 2 * D)
    p4 = _unpack(z3, c // 2, r)
    return p4.reshape(c, D, r).transpose(2, 0, 1)


# speedup vs baseline: 13.5318x; 1.1741x over previous
import functools

import jax
import jax.numpy as jnp
from jax import lax
from jax.experimental import pallas as pl
from jax.experimental.pallas import tpu as pltpu
from jax.experimental.pallas import tpu_sc as plsc

D = 64
NC, NS = 2, 16
NW = NC * NS
CHUNK = 128
NBUF = 4
SHIFT = 14

PB = 16384
CB = 2 * PB
UPR = 4096


def _repack_body(t_ref, o_ref):
    blk = t_ref[...]
    a = blk[:, :PB].T
    b = blk[:, PB:].T
    o_ref[...] = jnp.concatenate([a, b], axis=1)


def _repack(table_t, vocab):
    nblk = pl.cdiv(vocab, CB)
    out2 = pl.pallas_call(
        _repack_body,
        grid=(nblk,),
        in_specs=[pl.BlockSpec((D, CB), lambda i: (0, i))],
        out_specs=pl.BlockSpec((PB, 128), lambda i: (i, 0)),
        out_shape=jax.ShapeDtypeStruct((nblk * PB, 128), jnp.float32),
    )(table_t)
    return out2.reshape(nblk * PB * 2, D)


def _unpack_body(z_ref, p_ref):
    blk = z_ref[0]
    for s in range(UPR // 128):
        c = blk[128 * s : 128 * (s + 1), :].T
        p_ref[:, 0, :, 128 * s : 128 * (s + 1)] = c.reshape(2, D, 128)


def _unpack(z3, half, rows):
    nw = rows // UPR
    return pl.pallas_call(
        _unpack_body,
        grid=(half, nw),
        in_specs=[pl.BlockSpec((1, UPR, 128), lambda j, w: (j, w, 0))],
        out_specs=pl.BlockSpec((2, 1, D, UPR), lambda j, w: (0, j, 0, w)),
        out_shape=jax.ShapeDtypeStruct((2, half, D, rows), jnp.float32),
    )(z3)


@functools.lru_cache(maxsize=None)
def _make_gather(B, vpad):
    assert B % (NW * CHUNK * NBUF) == 0
    b_per_w = B // NW
    n_chunks = b_per_w // CHUNK
    mesh = plsc.VectorSubcoreMesh(core_axis_name="c", subcore_axis_name="s")

    @functools.partial(
        pl.kernel,
        mesh=mesh,
        out_type=jax.ShapeDtypeStruct((B, D), jnp.float32),
        compiler_params=pltpu.CompilerParams(use_tc_tiling_on_sc=False),
        scratch_types=[
            pltpu.VMEM((b_per_w,), jnp.int32),
            pltpu.VMEM((n_chunks, CHUNK), jnp.int32),
            pltpu.VMEM((NBUF, CHUNK, D), jnp.float32),
            pltpu.SemaphoreType.DMA((NBUF,)),
            pltpu.SemaphoreType.DMA,
        ],
    )
    def k(idx_hbm, table_hbm, out_hbm, idx_v, pos_v, bufs, sems, sem_s):
        wid = lax.axis_index("s") * NC + lax.axis_index("c")
        base = wid * b_per_w
        pltpu.sync_copy(idx_hbm.at[pl.ds(base, b_per_w)], idx_v)

        lane = lax.iota(jnp.int32, 16)

        def remap(g, carry):
            v = idx_v[pl.ds(g * 16, 16)]
            lin = (v & ~(2 * PB - 1)) | ((v & (PB - 1)) << 1) | ((v >> SHIFT) & 1)
            idx_v[pl.ds(g * 16, 16)] = lin
            b = base + g * 16 + lane
            j = b >> 14
            i = b & 16383
            h = jnp.where(j >= 10, 1, 0)
            pos = (j - 10 * h) * 32768 + 2 * i + h
            pos_v[g // 8, pl.ds(16 * (g % 8), 16)] = pos
            return carry

        lax.fori_loop(0, b_per_w // 16, remap, 0)

        def gather(i, b):
            pltpu.make_async_copy(
                table_hbm.at[idx_v.at[pl.ds(i * CHUNK, CHUNK)]],
                bufs.at[b],
                sems.at[b],
            ).start()

        for b in range(NBUF):
            gather(b, b)

        def body(g, carry):
            c = g * NBUF
            for b in range(NBUF):
                i = c + b
                pltpu.make_async_copy(
                    table_hbm.at[idx_v.at[pl.ds(0, CHUNK)]],
                    bufs.at[b],
                    sems.at[b],
                ).wait()
                pltpu.async_copy(
                    bufs.at[b], out_hbm.at[pos_v.at[i]], sem_s
                ).wait()
                nxt = i + NBUF

                @pl.when(nxt < n_chunks)
                def _():
                    gather(nxt, b)

            return carry

        lax.fori_loop(0, n_chunks // NBUF, body, 0)

    return k


@jax.jit
def kernel(x, table):
    r, c = x.shape
    B = r * c
    vocab = table.shape[0]
    x_flat = x.T.reshape(B).astype(jnp.int32)
    table_lin = _repack(table.T, vocab)
    out = _make_gather(B, table_lin.shape[0])(x_flat, table_lin)
    z3 = out.reshape(c // 2, r, 2 * D)
    p4 = _unpack(z3, c // 2, r)
    return p4.reshape(c, D, r).transpose(2, 0, 1)


# speedup vs baseline: 13.8998x; 1.0272x over previous
import functools

import jax
import jax.numpy as jnp
from jax import lax
from jax.experimental import pallas as pl
from jax.experimental.pallas import tpu as pltpu
from jax.experimental.pallas import tpu_sc as plsc

D = 64
NC, NS = 2, 16
NW = NC * NS
CHUNK = 128
NBUF = 4
SHIFT = 14

PB = 16384
CB = 2 * PB
UPR = 8192


def _repack_body(t_ref, o_ref):
    blk = t_ref[...]
    a = blk[:, :PB].T
    b = blk[:, PB:].T
    o_ref[...] = jnp.concatenate([a, b], axis=1)


def _repack(table_t, vocab):
    nblk = pl.cdiv(vocab, CB)
    out2 = pl.pallas_call(
        _repack_body,
        grid=(nblk,),
        in_specs=[pl.BlockSpec((D, CB), lambda i: (0, i))],
        out_specs=pl.BlockSpec((PB, 128), lambda i: (i, 0)),
        out_shape=jax.ShapeDtypeStruct((nblk * PB, 128), jnp.float32),
    )(table_t)
    return out2.reshape(nblk * PB * 2, D)


def _unpack_body(z_ref, p_ref):
    blk = z_ref[0]
    for s in range(UPR // 128):
        c = blk[128 * s : 128 * (s + 1), :].T
        p_ref[:, 0, :, 128 * s : 128 * (s + 1)] = c.reshape(2, D, 128)


def _unpack(z3, half, rows):
    nw = rows // UPR
    return pl.pallas_call(
        _unpack_body,
        grid=(half, nw),
        in_specs=[pl.BlockSpec((1, UPR, 128), lambda j, w: (j, w, 0))],
        out_specs=pl.BlockSpec((2, 1, D, UPR), lambda j, w: (0, j, 0, w)),
        out_shape=jax.ShapeDtypeStruct((2, half, D, rows), jnp.float32),
    )(z3)


@functools.lru_cache(maxsize=None)
def _make_gather(B, vpad):
    assert B % (NW * CHUNK * NBUF) == 0
    b_per_w = B // NW
    n_chunks = b_per_w // CHUNK
    mesh = plsc.VectorSubcoreMesh(core_axis_name="c", subcore_axis_name="s")

    @functools.partial(
        pl.kernel,
        mesh=mesh,
        out_type=jax.ShapeDtypeStruct((B, D), jnp.float32),
        compiler_params=pltpu.CompilerParams(use_tc_tiling_on_sc=False),
        scratch_types=[
            pltpu.VMEM((b_per_w,), jnp.int32),
            pltpu.VMEM((n_chunks, CHUNK), jnp.int32),
            pltpu.VMEM((NBUF, CHUNK, D), jnp.float32),
            pltpu.SemaphoreType.DMA((NBUF,)),
            pltpu.SemaphoreType.DMA,
        ],
    )
    def k(idx_hbm, table_hbm, out_hbm, idx_v, pos_v, bufs, sems, sem_s):
        wid = lax.axis_index("s") * NC + lax.axis_index("c")
        base = wid * b_per_w
        pltpu.sync_copy(idx_hbm.at[pl.ds(base, b_per_w)], idx_v)

        lane = lax.iota(jnp.int32, 16)

        def remap(g, carry):
            v = idx_v[pl.ds(g * 16, 16)]
            lin = (v & ~(2 * PB - 1)) | ((v & (PB - 1)) << 1) | ((v >> SHIFT) & 1)
            idx_v[pl.ds(g * 16, 16)] = lin
            b = base + g * 16 + lane
            j = b >> 14
            i = b & 16383
            h = jnp.where(j >= 10, 1, 0)
            pos = (j - 10 * h) * 32768 + 2 * i + h
            pos_v[g // 8, pl.ds(16 * (g % 8), 16)] = pos
            return carry

        lax.fori_loop(0, b_per_w // 16, remap, 0)

        def gather(i, b):
            pltpu.make_async_copy(
                table_hbm.at[idx_v.at[pl.ds(i * CHUNK, CHUNK)]],
                bufs.at[b],
                sems.at[b],
            ).start()

        for b in range(NBUF):
            gather(b, b)

        def body(g, carry):
            c = g * NBUF
            for b in range(NBUF):
                i = c + b
                pltpu.make_async_copy(
                    table_hbm.at[idx_v.at[pl.ds(0, CHUNK)]],
                    bufs.at[b],
                    sems.at[b],
                ).wait()
                pltpu.async_copy(
                    bufs.at[b], out_hbm.at[pos_v.at[i]], sem_s
                ).wait()
                nxt = i + NBUF

                @pl.when(nxt < n_chunks)
                def _():
                    gather(nxt, b)

            return carry

        lax.fori_loop(0, n_chunks // NBUF, body, 0)

    return k


@jax.jit
def kernel(x, table):
    r, c = x.shape
    B = r * c
    vocab = table.shape[0]
    x_flat = x.T.reshape(B).astype(jnp.int32)
    table_lin = _repack(table.T, vocab)
    out = _make_gather(B, table_lin.shape[0])(x_flat, table_lin)
    z3 = out.reshape(c // 2, r, 2 * D)
    p4 = _unpack(z3, c // 2, r)
    return p4.reshape(c, D, r).transpose(2, 0, 1)


# speedup vs baseline: 15.8852x; 1.1428x over previous
import functools

import jax
import jax.numpy as jnp
from jax import lax
from jax.experimental import pallas as pl
from jax.experimental.pallas import tpu as pltpu
from jax.experimental.pallas import tpu_sc as plsc

D = 64
NC, NS = 2, 16
NW = NC * NS
CHUNK = 128
NBUF = 4
SHIFT = 7

PAIR = 128
CB = 16384
UPR = 8192


def _repack_body(t_ref, o_ref):
    blk = t_ref[...]
    for s in range(CB // 256):
        a = blk[:, 256 * s : 256 * s + 128]
        b = blk[:, 256 * s + 128 : 256 * s + 256]
        c = jnp.concatenate([a, b], axis=0).T
        o_ref[128 * s : 128 * (s + 1), :] = c


def _repack(table_t, vocab):
    nblk = pl.cdiv(vocab, CB)
    out2 = pl.pallas_call(
        _repack_body,
        grid=(nblk,),
        in_specs=[pl.BlockSpec((D, CB), lambda i: (0, i))],
        out_specs=pl.BlockSpec((CB // 2, 128), lambda i: (i, 0)),
        out_shape=jax.ShapeDtypeStruct((nblk * CB // 2, 128), jnp.float32),
    )(table_t)
    return out2.reshape(nblk * CB, D)


def _unpack_body(z_ref, p_ref):
    blk = z_ref[0]
    for s in range(UPR // 128):
        c = blk[128 * s : 128 * (s + 1), :].T
        p_ref[:, 0, :, 128 * s : 128 * (s + 1)] = c.reshape(2, D, 128)


def _unpack(z3, half, rows):
    nw = rows // UPR
    return pl.pallas_call(
        _unpack_body,
        grid=(half, nw),
        in_specs=[pl.BlockSpec((1, UPR, 128), lambda j, w: (j, w, 0))],
        out_specs=pl.BlockSpec((2, 1, D, UPR), lambda j, w: (0, j, 0, w)),
        out_shape=jax.ShapeDtypeStruct((2, half, D, rows), jnp.float32),
    )(z3)


@functools.lru_cache(maxsize=None)
def _make_gather(B, vpad):
    assert B % (NW * CHUNK * NBUF) == 0
    b_per_w = B // NW
    n_chunks = b_per_w // CHUNK
    mesh = plsc.VectorSubcoreMesh(core_axis_name="c", subcore_axis_name="s")

    @functools.partial(
        pl.kernel,
        mesh=mesh,
        out_type=jax.ShapeDtypeStruct((B, D), jnp.float32),
        compiler_params=pltpu.CompilerParams(use_tc_tiling_on_sc=False),
        scratch_types=[
            pltpu.VMEM((b_per_w,), jnp.int32),
            pltpu.VMEM((n_chunks, CHUNK), jnp.int32),
            pltpu.VMEM((NBUF, CHUNK, D), jnp.float32),
            pltpu.SemaphoreType.DMA((NBUF,)),
            pltpu.SemaphoreType.DMA,
        ],
    )
    def k(idx_hbm, table_hbm, out_hbm, idx_v, pos_v, bufs, sems, sem_s):
        wid = lax.axis_index("s") * NC + lax.axis_index("c")
        base = wid * b_per_w
        pltpu.sync_copy(idx_hbm.at[pl.ds(base, b_per_w)], idx_v)

        lane = lax.iota(jnp.int32, 16)

        def remap(g, carry):
            v = idx_v[pl.ds(g * 16, 16)]
            lin = (v & ~(2 * PAIR - 1)) | ((v & (PAIR - 1)) << 1) | ((v >> SHIFT) & 1)
            idx_v[pl.ds(g * 16, 16)] = lin
            b = base + g * 16 + lane
            j = b >> 14
            i = b & 16383
            h = jnp.where(j >= 10, 1, 0)
            pos = (j - 10 * h) * 32768 + 2 * i + h
            pos_v[g // 8, pl.ds(16 * (g % 8), 16)] = pos
            return carry

        lax.fori_loop(0, b_per_w // 16, remap, 0)

        def gather(i, b):
            pltpu.make_async_copy(
                table_hbm.at[idx_v.at[pl.ds(i * CHUNK, CHUNK)]],
                bufs.at[b],
                sems.at[b],
            ).start()

        for b in range(NBUF):
            gather(b, b)

        def body(g, carry):
            c = g * NBUF
            for b in range(NBUF):
                i = c + b
                pltpu.make_async_copy(
                    table_hbm.at[idx_v.at[pl.ds(0, CHUNK)]],
                    bufs.at[b],
                    sems.at[b],
                ).wait()
                pltpu.async_copy(
                    bufs.at[b], out_hbm.at[pos_v.at[i]], sem_s
                ).wait()
                nxt = i + NBUF

                @pl.when(nxt < n_chunks)
                def _():
                    gather(nxt, b)

            return carry

        lax.fori_loop(0, n_chunks // NBUF, body, 0)

    return k


@jax.jit
def kernel(x, table):
    r, c = x.shape
    B = r * c
    vocab = table.shape[0]
    x_flat = x.T.reshape(B).astype(jnp.int32)
    table_lin = _repack(table.T, vocab)
    out = _make_gather(B, table_lin.shape[0])(x_flat, table_lin)
    z3 = out.reshape(c // 2, r, 2 * D)
    p4 = _unpack(z3, c // 2, r)
    return p4.reshape(c, D, r).transpose(2, 0, 1)


# speedup vs baseline: 16.1580x; 1.0172x over previous
import functools

import jax
import jax.numpy as jnp
from jax import lax
from jax.experimental import pallas as pl
from jax.experimental.pallas import tpu as pltpu
from jax.experimental.pallas import tpu_sc as plsc

D = 64
NC, NS = 2, 16
NW = NC * NS
CHUNK = 128
NBUF = 4
SHIFT = 7

PAIR = 128
CB = 32768
UPR = 8192


def _repack_body(t_ref, o_ref):
    blk = t_ref[...]
    for s in range(CB // 256):
        a = blk[:, 256 * s : 256 * s + 128]
        b = blk[:, 256 * s + 128 : 256 * s + 256]
        c = jnp.concatenate([a, b], axis=0).T
        o_ref[128 * s : 128 * (s + 1), :] = c


def _repack(table_t, vocab):
    nblk = pl.cdiv(vocab, CB)
    out2 = pl.pallas_call(
        _repack_body,
        grid=(nblk,),
        in_specs=[pl.BlockSpec((D, CB), lambda i: (0, i))],
        out_specs=pl.BlockSpec((CB // 2, 128), lambda i: (i, 0)),
        out_shape=jax.ShapeDtypeStruct((nblk * CB // 2, 128), jnp.float32),
    )(table_t)
    return out2.reshape(nblk * CB, D)


def _unpack_body(z_ref, p_ref):
    blk = z_ref[0]
    for s in range(UPR // 128):
        c = blk[128 * s : 128 * (s + 1), :].T
        p_ref[:, 0, :, 128 * s : 128 * (s + 1)] = c.reshape(2, D, 128)


def _unpack(z3, half, rows):
    nw = rows // UPR
    return pl.pallas_call(
        _unpack_body,
        grid=(half, nw),
        in_specs=[pl.BlockSpec((1, UPR, 128), lambda j, w: (j, w, 0))],
        out_specs=pl.BlockSpec((2, 1, D, UPR), lambda j, w: (0, j, 0, w)),
        out_shape=jax.ShapeDtypeStruct((2, half, D, rows), jnp.float32),
    )(z3)


@functools.lru_cache(maxsize=None)
def _make_gather(B, vpad):
    assert B % (NW * CHUNK * NBUF) == 0
    b_per_w = B // NW
    n_chunks = b_per_w // CHUNK
    mesh = plsc.VectorSubcoreMesh(core_axis_name="c", subcore_axis_name="s")

    @functools.partial(
        pl.kernel,
        mesh=mesh,
        out_type=jax.ShapeDtypeStruct((B, D), jnp.float32),
        compiler_params=pltpu.CompilerParams(use_tc_tiling_on_sc=False),
        scratch_types=[
            pltpu.VMEM((b_per_w,), jnp.int32),
            pltpu.VMEM((n_chunks, CHUNK), jnp.int32),
            pltpu.VMEM((NBUF, CHUNK, D), jnp.float32),
            pltpu.SemaphoreType.DMA((NBUF,)),
            pltpu.SemaphoreType.DMA,
        ],
    )
    def k(idx_hbm, table_hbm, out_hbm, idx_v, pos_v, bufs, sems, sem_s):
        wid = lax.axis_index("s") * NC + lax.axis_index("c")
        base = wid * b_per_w
        pltpu.sync_copy(idx_hbm.at[pl.ds(base, b_per_w)], idx_v)

        lane = lax.iota(jnp.int32, 16)

        def remap(g, carry):
            v = idx_v[pl.ds(g * 16, 16)]
            lin = (v & ~(2 * PAIR - 1)) | ((v & (PAIR - 1)) << 1) | ((v >> SHIFT) & 1)
            idx_v[pl.ds(g * 16, 16)] = lin
            b = base + g * 16 + lane
            j = b >> 14
            i = b & 16383
            h = jnp.where(j >= 10, 1, 0)
            pos = (j - 10 * h) * 32768 + 2 * i + h
            pos_v[g // 8, pl.ds(16 * (g % 8), 16)] = pos
            return carry

        lax.fori_loop(0, b_per_w // 16, remap, 0)

        def gather(i, b):
            pltpu.make_async_copy(
                table_hbm.at[idx_v.at[pl.ds(i * CHUNK, CHUNK)]],
                bufs.at[b],
                sems.at[b],
            ).start()

        for b in range(NBUF):
            gather(b, b)

        def body(g, carry):
            c = g * NBUF
            for b in range(NBUF):
                i = c + b
                pltpu.make_async_copy(
                    table_hbm.at[idx_v.at[pl.ds(0, CHUNK)]],
                    bufs.at[b],
                    sems.at[b],
                ).wait()
                pltpu.async_copy(
                    bufs.at[b], out_hbm.at[pos_v.at[i]], sem_s
                ).wait()
                nxt = i + NBUF

                @pl.when(nxt < n_chunks)
                def _():
                    gather(nxt, b)

            return carry

        lax.fori_loop(0, n_chunks // NBUF, body, 0)

    return k


@jax.jit
def kernel(x, table):
    r, c = x.shape
    B = r * c
    vocab = table.shape[0]
    x_flat = x.T.reshape(B).astype(jnp.int32)
    table_lin = _repack(table.T, vocab)
    out = _make_gather(B, table_lin.shape[0])(x_flat, table_lin)
    z3 = out.reshape(c // 2, r, 2 * D)
    p4 = _unpack(z3, c // 2, r)
    return p4.reshape(c, D, r).transpose(2, 0, 1)
